# Initial kernel scaffold; baseline (speedup 1.0000x reference)
import jax
import jax.numpy as jnp
from jax.experimental import pallas as pl


def kernel(user_ids, book_ids, user_table, book_table, W0, b0, W1, b1, W2, b2, W3, b3):
    raise NotImplementedError("write your pallas kernel here")


# speedup vs baseline: 2.1536x; 2.1536x over previous
import functools

import jax
import jax.numpy as jnp
from jax import lax
from jax.experimental import pallas as pl
from jax.experimental.pallas import tpu as pltpu
from jax.experimental.pallas import tpu_sc as plsc

_EMBED = 128
_BATCH = 16384
_NC = 2
_NS = 16
_NW = _NC * _NS
_B_PER_W = _BATCH // _NW


def _gather_body(user_tab, book_tab, uids, bids, u_out, b_out, idx_v, rows_v, sem):
    wid = lax.axis_index("s") * _NC + lax.axis_index("c")
    base = wid * _B_PER_W
    pltpu.sync_copy(uids.at[pl.ds(base, _B_PER_W)], idx_v)
    pltpu.async_copy(user_tab.at[idx_v], rows_v, sem).wait()
    pltpu.sync_copy(rows_v, u_out.at[pl.ds(base, _B_PER_W)])
    pltpu.sync_copy(bids.at[pl.ds(base, _B_PER_W)], idx_v)
    pltpu.async_copy(book_tab.at[idx_v], rows_v, sem).wait()
    pltpu.sync_copy(rows_v, b_out.at[pl.ds(base, _B_PER_W)])


_sc_gather = functools.partial(
    pl.kernel,
    mesh=plsc.VectorSubcoreMesh(core_axis_name="c", subcore_axis_name="s"),
    out_type=[
        jax.ShapeDtypeStruct((_BATCH, _EMBED), jnp.float32),
        jax.ShapeDtypeStruct((_BATCH, _EMBED), jnp.float32),
    ],
    scratch_types=[
        pltpu.VMEM((_B_PER_W,), jnp.int32),
        pltpu.VMEM((_B_PER_W, _EMBED), jnp.float32),
        pltpu.SemaphoreType.DMA,
    ],
)(_gather_body)


_BLK = 2048


def _mlp_body(u_ref, b_ref, w0u_ref, w0b_ref, b0_ref, w1_ref, b1_ref,
              w2_ref, b2_ref, w3_ref, out_ref):
    f32 = jnp.float32
    h = jnp.dot(u_ref[...], w0u_ref[...], preferred_element_type=f32)
    h += jnp.dot(b_ref[...], w0b_ref[...], preferred_element_type=f32)
    h = jnp.maximum(h + b0_ref[...], 0.0)
    h = jnp.maximum(jnp.dot(h, w1_ref[...], preferred_element_type=f32) + b1_ref[...], 0.0)
    h = jnp.maximum(jnp.dot(h, w2_ref[...], preferred_element_type=f32) + b2_ref[...], 0.0)
    out_ref[...] = jnp.dot(h, w3_ref[...], preferred_element_type=f32)


def _mlp(u, b, w0u, w0b, b0, w1, b1, w2, b2, w3):
    grid = _BATCH // _BLK
    full = lambda shape: pl.BlockSpec(shape, lambda i: (0,) * len(shape))
    return pl.pallas_call(
        _mlp_body,
        grid=(grid,),
        in_specs=[
            pl.BlockSpec((_BLK, _EMBED), lambda i: (i, 0)),
            pl.BlockSpec((_BLK, _EMBED), lambda i: (i, 0)),
            full((_EMBED, 512)),
            full((_EMBED, 512)),
            full((512,)),
            full((512, 256)),
            full((256,)),
            full((256, 128)),
            full((128,)),
            full((128,)),
        ],
        out_specs=pl.BlockSpec((_BLK,), lambda i: (i,)),
        out_shape=jax.ShapeDtypeStruct((_BATCH,), jnp.float32),
    )(u, b, w0u, w0b, b0, w1, b1, w2, b2, w3)


@jax.jit
def kernel(user_ids, book_ids, user_table, book_table,
           W0, b0, W1, b1, W2, b2, W3, b3):
    u, b = _sc_gather(user_table, book_table,
                      user_ids.astype(jnp.int32), book_ids.astype(jnp.int32))
    out = _mlp(u, b, W0[:_EMBED], W0[_EMBED:], b0, W1, b1, W2, b2,
               W3.reshape(_EMBED))
    return out + b3[0]


# speedup vs baseline: 2.6609x; 1.2356x over previous
import functools

import jax
import jax.numpy as jnp
from jax import lax
from jax.experimental import pallas as pl
from jax.experimental.pallas import tpu as pltpu
from jax.experimental.pallas import tpu_sc as plsc

_EMBED = 128
_BATCH = 16384
_NC = 2
_NS = 16
_NW = _NC * _NS
_B_PER_W = _BATCH // _NW


def _gather_body(user_tab, book_tab, uids, bids, u_out, b_out, idx_v, rows_v, sem):
    wid = lax.axis_index("s") * _NC + lax.axis_index("c")
    base = wid * _B_PER_W
    pltpu.sync_copy(uids.at[pl.ds(base, _B_PER_W)], idx_v)
    pltpu.async_copy(user_tab.at[idx_v], rows_v, sem).wait()
    pltpu.sync_copy(rows_v, u_out.at[pl.ds(base, _B_PER_W)])
    pltpu.sync_copy(bids.at[pl.ds(base, _B_PER_W)], idx_v)
    pltpu.async_copy(book_tab.at[idx_v], rows_v, sem).wait()
    pltpu.sync_copy(rows_v, b_out.at[pl.ds(base, _B_PER_W)])


_sc_gather = functools.partial(
    pl.kernel,
    mesh=plsc.VectorSubcoreMesh(core_axis_name="c", subcore_axis_name="s"),
    out_type=[
        jax.ShapeDtypeStruct((_BATCH, _EMBED), jnp.float32),
        jax.ShapeDtypeStruct((_BATCH, _EMBED), jnp.float32),
    ],
    scratch_types=[
        pltpu.VMEM((_B_PER_W,), jnp.int32),
        pltpu.VMEM((_B_PER_W, _EMBED), jnp.float32),
        pltpu.SemaphoreType.DMA,
    ],
)(_gather_body)


_BLK = 2048


def _mlp_body(u_ref, b_ref, w0u_ref, w0b_ref, b0_ref, w1_ref, b1_ref,
              w2_ref, b2_ref, w3_ref, out_ref):
    f32, bf16 = jnp.float32, jnp.bfloat16
    h = jnp.dot(u_ref[...].astype(bf16), w0u_ref[...], preferred_element_type=f32)
    h += jnp.dot(b_ref[...].astype(bf16), w0b_ref[...], preferred_element_type=f32)
    h = jnp.maximum(h + b0_ref[...], 0.0).astype(bf16)
    h = jnp.maximum(jnp.dot(h, w1_ref[...], preferred_element_type=f32) + b1_ref[...], 0.0).astype(bf16)
    h = jnp.maximum(jnp.dot(h, w2_ref[...], preferred_element_type=f32) + b2_ref[...], 0.0).astype(bf16)
    out_ref[...] = jnp.dot(h, w3_ref[...], preferred_element_type=f32)


def _mlp(u, b, w0u, w0b, b0, w1, b1, w2, b2, w3):
    grid = _BATCH // _BLK
    full = lambda shape: pl.BlockSpec(shape, lambda i: (0,) * len(shape))
    return pl.pallas_call(
        _mlp_body,
        grid=(grid,),
        in_specs=[
            pl.BlockSpec((_BLK, _EMBED), lambda i: (i, 0)),
            pl.BlockSpec((_BLK, _EMBED), lambda i: (i, 0)),
            full((_EMBED, 512)),
            full((_EMBED, 512)),
            full((512,)),
            full((512, 256)),
            full((256,)),
            full((256, 128)),
            full((128,)),
            full((128,)),
        ],
        out_specs=pl.BlockSpec((_BLK,), lambda i: (i,)),
        out_shape=jax.ShapeDtypeStruct((_BATCH,), jnp.float32),
    )(u, b, w0u, w0b, b0, w1, b1, w2, b2, w3)


@jax.jit
def kernel(user_ids, book_ids, user_table, book_table,
           W0, b0, W1, b1, W2, b2, W3, b3):
    u, b = _sc_gather(user_table, book_table,
                      user_ids.astype(jnp.int32), book_ids.astype(jnp.int32))
    bf16 = jnp.bfloat16
    out = _mlp(u, b, W0[:_EMBED].astype(bf16), W0[_EMBED:].astype(bf16), b0,
               W1.astype(bf16), b1, W2.astype(bf16), b2,
               W3.reshape(_EMBED).astype(bf16))
    return out + b3[0]


# speedup vs baseline: 3.2908x; 1.2367x over previous
import functools

import jax
import jax.numpy as jnp
from jax import lax
from jax.experimental import pallas as pl
from jax.experimental.pallas import tpu as pltpu
from jax.experimental.pallas import tpu_sc as plsc

_EMBED = 128
_BATCH = 16384
_NC = 2
_NS = 16
_NW = _NC * _NS
_B_PER_W = _BATCH // _NW


def _gather_body(user_tab, book_tab, uids, bids, u_out, b_out, idx_v, rows_v, sem):
    wid = lax.axis_index("s") * _NC + lax.axis_index("c")
    base = wid * _B_PER_W
    pltpu.sync_copy(uids.at[pl.ds(base, _B_PER_W)], idx_v)
    pltpu.async_copy(user_tab.at[idx_v], rows_v, sem).wait()
    pltpu.sync_copy(rows_v, u_out.at[pl.ds(base, _B_PER_W)])
    pltpu.sync_copy(bids.at[pl.ds(base, _B_PER_W)], idx_v)
    pltpu.async_copy(book_tab.at[idx_v], rows_v, sem).wait()
    pltpu.sync_copy(rows_v, b_out.at[pl.ds(base, _B_PER_W)])


_sc_gather = functools.partial(
    pl.kernel,
    mesh=plsc.VectorSubcoreMesh(core_axis_name="c", subcore_axis_name="s"),
    out_type=[
        jax.ShapeDtypeStruct((_BATCH, _EMBED), jnp.float32),
        jax.ShapeDtypeStruct((_BATCH, _EMBED), jnp.float32),
    ],
    scratch_types=[
        pltpu.VMEM((_B_PER_W,), jnp.int32),
        pltpu.VMEM((_B_PER_W, _EMBED), jnp.float32),
        pltpu.SemaphoreType.DMA,
    ],
)(_gather_body)


_BLK = 2048

_DNUMS = (((0,), (0,)), ((), ()))


def _mlp_body(u_ref, b_ref, w0_ref, b0_ref, w1_ref, b1_ref,
              w2_ref, b2_ref, w3_ref, b3_ref, out_ref, xt_ref):
    f32, bf16 = jnp.float32, jnp.bfloat16
    dot = functools.partial(lax.dot_general, dimension_numbers=_DNUMS,
                            preferred_element_type=f32)
    xt_ref[:_EMBED, :] = u_ref[...].astype(bf16).T
    xt_ref[_EMBED:, :] = b_ref[...].astype(bf16).T
    h = dot(w0_ref[...].astype(bf16), xt_ref[...])
    h = jnp.maximum(h + b0_ref[...], 0.0).astype(bf16)
    h = dot(w1_ref[...].astype(bf16), h)
    h = jnp.maximum(h + b1_ref[...], 0.0).astype(bf16)
    h = dot(w2_ref[...].astype(bf16), h)
    h = jnp.maximum(h + b2_ref[...], 0.0).astype(bf16)
    y = dot(w3_ref[...].astype(bf16), h)
    out_ref[...] = y[0] + b3_ref[0]


def _mlp(u, b, w0, b0, w1, b1, w2, b2, w3, b3):
    grid = _BATCH // _BLK
    full = lambda shape: pl.BlockSpec(shape, lambda i: (0,) * len(shape))
    return pl.pallas_call(
        _mlp_body,
        grid=(grid,),
        in_specs=[
            pl.BlockSpec((_BLK, _EMBED), lambda i: (i, 0)),
            pl.BlockSpec((_BLK, _EMBED), lambda i: (i, 0)),
            full((2 * _EMBED, 512)),
            full((512, 1)),
            full((512, 256)),
            full((256, 1)),
            full((256, 128)),
            full((128, 1)),
            full((_EMBED, 1)),
            full((1,)),
        ],
        out_specs=pl.BlockSpec((_BLK,), lambda i: (i,)),
        out_shape=jax.ShapeDtypeStruct((_BATCH,), jnp.float32),
        scratch_shapes=[pltpu.VMEM((2 * _EMBED, _BLK), jnp.bfloat16)],
    )(u, b, w0, b0, w1, b1, w2, b2, w3, b3)


@jax.jit
def kernel(user_ids, book_ids, user_table, book_table,
           W0, b0, W1, b1, W2, b2, W3, b3):
    u, b = _sc_gather(user_table, book_table,
                      user_ids.astype(jnp.int32), book_ids.astype(jnp.int32))
    return _mlp(u, b, W0, b0.reshape(-1, 1), W1, b1.reshape(-1, 1),
                W2, b2.reshape(-1, 1), W3, b3)
